# Initial kernel scaffold; baseline (speedup 1.0000x reference)
import jax
import jax.numpy as jnp
from jax.experimental import pallas as pl


def kernel(xyz, W1, b1, g1, beta1, W2, b2, W3, b3, g2, beta2, W4, b4):
    raise NotImplementedError("write your pallas kernel here")


# speedup vs baseline: 220.1474x; 220.1474x over previous
import functools

import jax
import jax.numpy as jnp
from jax import lax
from jax.experimental import pallas as pl
from jax.experimental.pallas import tpu as pltpu
from jax.experimental.pallas import tpu_sc as plsc

B, N, K, ITERS = 16, 4096, 256, 10
D1, D2, D3, D4 = 128, 256, 512, 256
HALF = D2 // 2
CH = 512


def _stage1_body(x8_ref, x8t_ref, w1_ref, b1_ref, g1_ref, be1_ref,
                 w2_ref, b2_ref, cent_ref, lab_ref, pi_ref, h_ref):
    x8 = x8_ref[0]
    x8t = x8t_ref[0]
    iota8r = lax.broadcasted_iota(jnp.int32, (1, 8), 1)
    iota8c = lax.broadcasted_iota(jnp.int32, (8, 1), 0)
    cmask_r = (iota8r < 3).astype(jnp.float32)
    cmask_c = (iota8c < 3).astype(jnp.float32)
    e3_c = (iota8c == 3).astype(jnp.float32)
    x2 = jnp.sum(x8 * x8 * cmask_r, axis=1, keepdims=True)
    iota_k = lax.broadcasted_iota(jnp.int32, (1, K), 1)

    centT0 = (x8[:K, :] * cmask_r).T

    def one_iter(_, carry):
        centT, _labels = carry
        t = jnp.dot(x8.astype(jnp.bfloat16), centT.astype(jnp.bfloat16),
                    preferred_element_type=jnp.float32)
        c2 = jnp.sum(centT * centT, axis=0, keepdims=True)
        d2 = (x2 - 2.0 * t) + c2
        m = jnp.min(d2, axis=1, keepdims=True)
        labels = jnp.min(jnp.where(d2 == m, iota_k, K), axis=1)
        labels = labels.astype(jnp.int32)
        oh = (labels[:, None] == iota_k).astype(jnp.float32)
        statsT = jnp.dot(x8t.astype(jnp.bfloat16), oh.astype(jnp.bfloat16),
                         preferred_element_type=jnp.float32)
        counts = jnp.sum(statsT * e3_c, axis=0, keepdims=True)
        centT_new = statsT * cmask_c / jnp.maximum(counts, 1.0)
        return (centT_new, labels)

    labels0 = jnp.zeros((N,), dtype=jnp.int32)
    centT, labels = lax.fori_loop(0, ITERS, one_iter, (centT0, labels0))

    oh = (labels[:, None] == iota_k).astype(jnp.float32)
    cent8 = centT.T
    p_i8 = jnp.dot(oh, cent8, preferred_element_type=jnp.float32,
                   precision=lax.Precision.HIGHEST)
    rel8 = x8 * cmask_r - p_i8

    h1 = jnp.dot(rel8, w1_ref[...], preferred_element_type=jnp.float32)
    h1 = h1 + b1_ref[...]
    mu = jnp.mean(h1, axis=1, keepdims=True)
    var = jnp.mean((h1 - mu) ** 2, axis=1, keepdims=True)
    hn = (h1 - mu) / jnp.sqrt(var + 1e-5) * g1_ref[...] + be1_ref[...]
    h = jnp.dot(jnp.maximum(hn, 0.0), w2_ref[...],
                preferred_element_type=jnp.float32) + b2_ref[...]

    cent_ref[0] = cent8[:, :3]
    lab_ref[0, 0] = labels
    pi_ref[0] = p_i8[:, :3]
    h_ref[0] = h


def _stage1(x8, x8t, w1p, b1, g1, be1, w2, b2):
    return pl.pallas_call(
        _stage1_body,
        grid=(B,),
        in_specs=[
            pl.BlockSpec((1, N, 8), lambda b: (b, 0, 0)),
            pl.BlockSpec((1, 8, N), lambda b: (b, 0, 0)),
            pl.BlockSpec((8, D1), lambda b: (0, 0)),
            pl.BlockSpec((1, D1), lambda b: (0, 0)),
            pl.BlockSpec((1, D1), lambda b: (0, 0)),
            pl.BlockSpec((1, D1), lambda b: (0, 0)),
            pl.BlockSpec((D1, D2), lambda b: (0, 0)),
            pl.BlockSpec((1, D2), lambda b: (0, 0)),
        ],
        out_specs=[
            pl.BlockSpec((1, K, 3), lambda b: (b, 0, 0)),
            pl.BlockSpec((1, 1, N), lambda b: (b, 0, 0)),
            pl.BlockSpec((1, N, 3), lambda b: (b, 0, 0)),
            pl.BlockSpec((1, N, D2), lambda b: (b, 0, 0)),
        ],
        out_shape=[
            jax.ShapeDtypeStruct((B, K, 3), jnp.float32),
            jax.ShapeDtypeStruct((B, 1, N), jnp.int32),
            jax.ShapeDtypeStruct((B, N, 3), jnp.float32),
            jax.ShapeDtypeStruct((B, N, D2), jnp.float32),
        ],
        compiler_params=pltpu.CompilerParams(
            dimension_semantics=("arbitrary",)),
    )(x8, x8t, w1p, b1, g1, be1, w2, b2)


def _segmax_sc(data, labels):
    c = data.shape[-1]
    half = c // 2
    mesh = plsc.VectorSubcoreMesh(core_axis_name="c", subcore_axis_name="s")

    @functools.partial(
        pl.kernel,
        out_type=jax.ShapeDtypeStruct((B, K, c), jnp.float32),
        mesh=mesh,
        scratch_types=[
            pltpu.VMEM((N,), jnp.int32),
            pltpu.VMEM((CH, half), jnp.float32),
            pltpu.VMEM((K, half), jnp.float32),
        ],
    )
    def seg(data_hbm, lab_hbm, out_hbm, lab_v, buf, acc):
        b = lax.axis_index("s")
        col0 = lax.axis_index("c") * half
        pltpu.sync_copy(lab_hbm.at[b], lab_v)

        neg = jnp.full((16,), -jnp.inf, dtype=jnp.float32)

        def init_body(k, carry):
            for j in range(half // 16):
                acc[k, pl.ds(j * 16, 16)] = neg
            return carry
        lax.fori_loop(0, K, init_body, 0)

        def chunk_body(g, carry):
            r0 = g * CH
            pltpu.sync_copy(data_hbm.at[b, pl.ds(r0, CH), pl.ds(col0, half)],
                            buf)

            def row_body(rg, c2):
                lv = lab_v[pl.ds(r0 + rg * 16, 16)]
                for i in range(16):
                    l = lv[i]
                    r = rg * 16 + i
                    for j in range(half // 16):
                        a = acc[l, pl.ds(j * 16, 16)]
                        v = buf[r, pl.ds(j * 16, 16)]
                        acc[l, pl.ds(j * 16, 16)] = jnp.maximum(a, v)
                return c2
            lax.fori_loop(0, CH // 16, row_body, 0)
            return carry
        lax.fori_loop(0, N // CH, chunk_body, 0)

        def fix_body(k, carry):
            for j in range(half // 16):
                v = acc[k, pl.ds(j * 16, 16)]
                acc[k, pl.ds(j * 16, 16)] = jnp.where(v == -jnp.inf, 0.0, v)
            return carry
        lax.fori_loop(0, K, fix_body, 0)

        pltpu.sync_copy(acc, out_hbm.at[b, :, pl.ds(col0, half)])

    return seg(data, labels)


def _stage2_body(h_ref, lab_ref, pool_ref, w3a_ref, w3b_ref, b3_ref,
                 g2_ref, be2_ref, w4_ref, b4_ref, h2_ref):
    labels = lab_ref[0, 0]
    iota_k = lax.broadcasted_iota(jnp.int32, (1, K), 1)
    oh = (labels[:, None] == iota_k).astype(jnp.float32)
    reapted = jnp.dot(oh, pool_ref[0], preferred_element_type=jnp.float32,
                      precision=lax.Precision.HIGHEST)
    z = (jnp.dot(reapted, w3a_ref[...], preferred_element_type=jnp.float32)
         + jnp.dot(h_ref[0], w3b_ref[...], preferred_element_type=jnp.float32)
         + b3_ref[...])
    mu = jnp.mean(z, axis=1, keepdims=True)
    var = jnp.mean((z - mu) ** 2, axis=1, keepdims=True)
    zn = (z - mu) / jnp.sqrt(var + 1e-5) * g2_ref[...] + be2_ref[...]
    h2 = jnp.dot(jnp.maximum(zn, 0.0), w4_ref[...],
                 preferred_element_type=jnp.float32) + b4_ref[...]
    h2_ref[0] = h2


def _stage2(h, labels3, pooled, w3a, w3b, b3, g2, be2, w4, b4):
    return pl.pallas_call(
        _stage2_body,
        grid=(B,),
        in_specs=[
            pl.BlockSpec((1, N, D2), lambda b: (b, 0, 0)),
            pl.BlockSpec((1, 1, N), lambda b: (b, 0, 0)),
            pl.BlockSpec((1, K, D2), lambda b: (b, 0, 0)),
            pl.BlockSpec((D2, D3), lambda b: (0, 0)),
            pl.BlockSpec((D2, D3), lambda b: (0, 0)),
            pl.BlockSpec((1, D3), lambda b: (0, 0)),
            pl.BlockSpec((1, D3), lambda b: (0, 0)),
            pl.BlockSpec((1, D3), lambda b: (0, 0)),
            pl.BlockSpec((D3, D4), lambda b: (0, 0)),
            pl.BlockSpec((1, D4), lambda b: (0, 0)),
        ],
        out_specs=pl.BlockSpec((1, N, D4), lambda b: (b, 0, 0)),
        out_shape=jax.ShapeDtypeStruct((B, N, D4), jnp.float32),
        compiler_params=pltpu.CompilerParams(
            dimension_semantics=("arbitrary",)),
    )(h, labels3, pooled, w3a, w3b, b3, g2, be2, w4, b4)


def kernel(xyz, W1, b1, g1, beta1, W2, b2, W3, b3, g2, beta2, W4, b4):
    ones = jnp.ones((B, N, 1), jnp.float32)
    zeros = jnp.zeros((B, N, 4), jnp.float32)
    x8 = jnp.concatenate([xyz, ones, zeros], axis=-1)
    x8t = jnp.transpose(x8, (0, 2, 1))
    w1p = jnp.concatenate([W1, jnp.zeros((5, D1), jnp.float32)], axis=0)

    cent, labels3, p_i, h = _stage1(
        x8, x8t, w1p, b1.reshape(1, D1), g1.reshape(1, D1),
        beta1.reshape(1, D1), W2, b2.reshape(1, D2))

    labels = labels3.reshape(B, N)
    pooled = _segmax_sc(h, labels)

    h2 = _stage2(h, labels3, pooled, W3[:D2], W3[D2:], b3.reshape(1, D3),
                 g2.reshape(1, D3), beta2.reshape(1, D3), W4,
                 b4.reshape(1, D4))

    out = _segmax_sc(h2, labels)
    return (cent, out, p_i, labels)


# speedup vs baseline: 223.4616x; 1.0151x over previous
import functools

import jax
import jax.numpy as jnp
from jax import lax
from jax.experimental import pallas as pl
from jax.experimental.pallas import tpu as pltpu
from jax.experimental.pallas import tpu_sc as plsc

B, N, K, ITERS = 16, 4096, 256, 10
D1, D2, D3, D4 = 128, 256, 512, 256
HALF = D2 // 2
CH = 256


def _stage1_body(x8_ref, x8t_ref, w1_ref, b1_ref, g1_ref, be1_ref,
                 w2_ref, b2_ref, cent_ref, lab_ref, pi_ref, h_ref):
    x8 = x8_ref[0]
    x8t = x8t_ref[0]
    iota8r = lax.broadcasted_iota(jnp.int32, (1, 8), 1)
    iota8c = lax.broadcasted_iota(jnp.int32, (8, 1), 0)
    cmask_r = (iota8r < 3).astype(jnp.float32)
    cmask_c = (iota8c < 3).astype(jnp.float32)
    e3_c = (iota8c == 3).astype(jnp.float32)
    x2 = jnp.sum(x8 * x8 * cmask_r, axis=1, keepdims=True)
    iota_k = lax.broadcasted_iota(jnp.int32, (1, K), 1)

    centT0 = (x8[:K, :] * cmask_r).T

    def one_iter(_, carry):
        centT, _labels = carry
        t = jnp.dot(x8.astype(jnp.bfloat16), centT.astype(jnp.bfloat16),
                    preferred_element_type=jnp.float32)
        c2 = jnp.sum(centT * centT, axis=0, keepdims=True)
        d2 = (x2 - 2.0 * t) + c2
        m = jnp.min(d2, axis=1, keepdims=True)
        labels = jnp.min(jnp.where(d2 == m, iota_k, K), axis=1)
        labels = labels.astype(jnp.int32)
        oh = (labels[:, None] == iota_k).astype(jnp.float32)
        statsT = jnp.dot(x8t.astype(jnp.bfloat16), oh.astype(jnp.bfloat16),
                         preferred_element_type=jnp.float32)
        counts = jnp.sum(statsT * e3_c, axis=0, keepdims=True)
        centT_new = statsT * cmask_c / jnp.maximum(counts, 1.0)
        return (centT_new, labels)

    labels0 = jnp.zeros((N,), dtype=jnp.int32)
    centT, labels = lax.fori_loop(0, ITERS, one_iter, (centT0, labels0))

    oh = (labels[:, None] == iota_k).astype(jnp.float32)
    cent8 = centT.T
    p_i8 = jnp.dot(oh, cent8, preferred_element_type=jnp.float32,
                   precision=lax.Precision.HIGHEST)
    rel8 = x8 * cmask_r - p_i8

    h1 = jnp.dot(rel8, w1_ref[...], preferred_element_type=jnp.float32)
    h1 = h1 + b1_ref[...]
    mu = jnp.mean(h1, axis=1, keepdims=True)
    var = jnp.mean((h1 - mu) ** 2, axis=1, keepdims=True)
    hn = (h1 - mu) / jnp.sqrt(var + 1e-5) * g1_ref[...] + be1_ref[...]
    h = jnp.dot(jnp.maximum(hn, 0.0), w2_ref[...],
                preferred_element_type=jnp.float32) + b2_ref[...]

    cent_ref[0] = cent8[:, :3]
    lab_ref[0, 0] = labels
    pi_ref[0] = p_i8[:, :3]
    h_ref[0] = h


def _stage1(x8, x8t, w1p, b1, g1, be1, w2, b2):
    return pl.pallas_call(
        _stage1_body,
        grid=(B,),
        in_specs=[
            pl.BlockSpec((1, N, 8), lambda b: (b, 0, 0)),
            pl.BlockSpec((1, 8, N), lambda b: (b, 0, 0)),
            pl.BlockSpec((8, D1), lambda b: (0, 0)),
            pl.BlockSpec((1, D1), lambda b: (0, 0)),
            pl.BlockSpec((1, D1), lambda b: (0, 0)),
            pl.BlockSpec((1, D1), lambda b: (0, 0)),
            pl.BlockSpec((D1, D2), lambda b: (0, 0)),
            pl.BlockSpec((1, D2), lambda b: (0, 0)),
        ],
        out_specs=[
            pl.BlockSpec((1, K, 3), lambda b: (b, 0, 0)),
            pl.BlockSpec((1, 1, N), lambda b: (b, 0, 0)),
            pl.BlockSpec((1, N, 3), lambda b: (b, 0, 0)),
            pl.BlockSpec((1, N, D2), lambda b: (b, 0, 0)),
        ],
        out_shape=[
            jax.ShapeDtypeStruct((B, K, 3), jnp.float32),
            jax.ShapeDtypeStruct((B, 1, N), jnp.int32),
            jax.ShapeDtypeStruct((B, N, 3), jnp.float32),
            jax.ShapeDtypeStruct((B, N, D2), jnp.float32),
        ],
        compiler_params=pltpu.CompilerParams(
            dimension_semantics=("arbitrary",)),
    )(x8, x8t, w1p, b1, g1, be1, w2, b2)


def _segmax_sc(data, labels):
    c = data.shape[-1]
    half = c // 2
    mesh = plsc.VectorSubcoreMesh(core_axis_name="c", subcore_axis_name="s")

    @functools.partial(
        pl.kernel,
        out_type=jax.ShapeDtypeStruct((B, K, c), jnp.float32),
        mesh=mesh,
    scratch_types=[
            pltpu.VMEM((N,), jnp.int32),
            pltpu.VMEM((CH, half), jnp.float32),
            pltpu.VMEM((K, half), jnp.float32),
            pltpu.VMEM((K, half), jnp.float32),
        ],
    )
    def seg(data_hbm, lab_hbm, out_hbm, lab_v, buf, acc0, acc1):
        b = lax.axis_index("s")
        col0 = lax.axis_index("c") * half
        pltpu.sync_copy(lab_hbm.at[b], lab_v)

        neg = jnp.full((16,), -jnp.inf, dtype=jnp.float32)

        def init_body(k, carry):
            for j in range(half // 16):
                acc0[k, pl.ds(j * 16, 16)] = neg
                acc1[k, pl.ds(j * 16, 16)] = neg
            return carry
        lax.fori_loop(0, K, init_body, 0)

        def chunk_body(g, carry):
            r0 = g * CH
            pltpu.sync_copy(data_hbm.at[b, pl.ds(r0, CH), pl.ds(col0, half)],
                            buf)

            def row_body(rg, c2):
                lv = lab_v[pl.ds(r0 + rg * 16, 16)]
                for i in range(0, 16, 2):
                    l0 = lv[i]
                    l1 = lv[i + 1]
                    r = rg * 16 + i
                    for j in range(half // 16):
                        a = acc0[l0, pl.ds(j * 16, 16)]
                        v = buf[r, pl.ds(j * 16, 16)]
                        acc0[l0, pl.ds(j * 16, 16)] = jnp.maximum(a, v)
                        a1 = acc1[l1, pl.ds(j * 16, 16)]
                        v1 = buf[r + 1, pl.ds(j * 16, 16)]
                        acc1[l1, pl.ds(j * 16, 16)] = jnp.maximum(a1, v1)
                return c2
            lax.fori_loop(0, CH // 16, row_body, 0)
            return carry
        lax.fori_loop(0, N // CH, chunk_body, 0)

        def fix_body(k, carry):
            for j in range(half // 16):
                v = jnp.maximum(acc0[k, pl.ds(j * 16, 16)],
                                acc1[k, pl.ds(j * 16, 16)])
                acc0[k, pl.ds(j * 16, 16)] = jnp.where(v == -jnp.inf, 0.0, v)
            return carry
        lax.fori_loop(0, K, fix_body, 0)

        pltpu.sync_copy(acc0, out_hbm.at[b, :, pl.ds(col0, half)])

    return seg(data, labels)


def _stage2_body(h_ref, lab_ref, pool_ref, w3a_ref, w3b_ref, b3_ref,
                 g2_ref, be2_ref, w4_ref, b4_ref, h2_ref):
    labels = lab_ref[0, 0]
    iota_k = lax.broadcasted_iota(jnp.int32, (1, K), 1)
    oh = (labels[:, None] == iota_k).astype(jnp.float32)
    reapted = jnp.dot(oh.astype(jnp.bfloat16),
                      pool_ref[0].astype(jnp.bfloat16),
                      preferred_element_type=jnp.float32)
    z = (jnp.dot(reapted, w3a_ref[...], preferred_element_type=jnp.float32)
         + jnp.dot(h_ref[0], w3b_ref[...], preferred_element_type=jnp.float32)
         + b3_ref[...])
    mu = jnp.mean(z, axis=1, keepdims=True)
    var = jnp.mean((z - mu) ** 2, axis=1, keepdims=True)
    zn = (z - mu) / jnp.sqrt(var + 1e-5) * g2_ref[...] + be2_ref[...]
    h2 = jnp.dot(jnp.maximum(zn, 0.0), w4_ref[...],
                 preferred_element_type=jnp.float32) + b4_ref[...]
    h2_ref[0] = h2


def _stage2(h, labels3, pooled, w3a, w3b, b3, g2, be2, w4, b4):
    return pl.pallas_call(
        _stage2_body,
        grid=(B,),
        in_specs=[
            pl.BlockSpec((1, N, D2), lambda b: (b, 0, 0)),
            pl.BlockSpec((1, 1, N), lambda b: (b, 0, 0)),
            pl.BlockSpec((1, K, D2), lambda b: (b, 0, 0)),
            pl.BlockSpec((D2, D3), lambda b: (0, 0)),
            pl.BlockSpec((D2, D3), lambda b: (0, 0)),
            pl.BlockSpec((1, D3), lambda b: (0, 0)),
            pl.BlockSpec((1, D3), lambda b: (0, 0)),
            pl.BlockSpec((1, D3), lambda b: (0, 0)),
            pl.BlockSpec((D3, D4), lambda b: (0, 0)),
            pl.BlockSpec((1, D4), lambda b: (0, 0)),
        ],
        out_specs=pl.BlockSpec((1, N, D4), lambda b: (b, 0, 0)),
        out_shape=jax.ShapeDtypeStruct((B, N, D4), jnp.float32),
        compiler_params=pltpu.CompilerParams(
            dimension_semantics=("arbitrary",)),
    )(h, labels3, pooled, w3a, w3b, b3, g2, be2, w4, b4)


def kernel(xyz, W1, b1, g1, beta1, W2, b2, W3, b3, g2, beta2, W4, b4):
    ones = jnp.ones((B, N, 1), jnp.float32)
    zeros = jnp.zeros((B, N, 4), jnp.float32)
    x8 = jnp.concatenate([xyz, ones, zeros], axis=-1)
    x8t = jnp.transpose(x8, (0, 2, 1))
    w1p = jnp.concatenate([W1, jnp.zeros((5, D1), jnp.float32)], axis=0)

    cent, labels3, p_i, h = _stage1(
        x8, x8t, w1p, b1.reshape(1, D1), g1.reshape(1, D1),
        beta1.reshape(1, D1), W2, b2.reshape(1, D2))

    labels = labels3.reshape(B, N)
    pooled = _segmax_sc(h, labels)

    h2 = _stage2(h, labels3, pooled, W3[:D2], W3[D2:], b3.reshape(1, D3),
                 g2.reshape(1, D3), beta2.reshape(1, D3), W4,
                 b4.reshape(1, D4))

    out = _segmax_sc(h2, labels)
    return (cent, out, p_i, labels)


# speedup vs baseline: 223.9391x; 1.0021x over previous
import functools

import jax
import jax.numpy as jnp
from jax import lax
from jax.experimental import pallas as pl
from jax.experimental.pallas import tpu as pltpu
from jax.experimental.pallas import tpu_sc as plsc

B, N, K, ITERS = 16, 4096, 256, 10
D1, D2, D3, D4 = 128, 256, 512, 256
HALF = D2 // 2
CH = 256


def _stage1_body(x8_ref, x8t_ref, w1_ref, b1_ref, g1_ref, be1_ref,
                 w2_ref, b2_ref, cent_ref, lab_ref, pi_ref, h_ref):
    x8 = x8_ref[0]
    x8t = x8t_ref[0]
    iota8r = lax.broadcasted_iota(jnp.int32, (1, 8), 1)
    iota8c = lax.broadcasted_iota(jnp.int32, (8, 1), 0)
    cmask_r = (iota8r < 3).astype(jnp.float32)
    cmask_c = (iota8c < 3).astype(jnp.float32)
    e3_c = (iota8c == 3).astype(jnp.float32)
    x2 = jnp.sum(x8 * x8 * cmask_r, axis=1, keepdims=True)
    iota_k = lax.broadcasted_iota(jnp.int32, (1, K), 1)

    centT0 = (x8[:K, :] * cmask_r).T
    x8b = x8.astype(jnp.bfloat16)
    x8tb = x8t.astype(jnp.bfloat16)

    def one_iter(_, carry):
        centT, _labels = carry
        t = jnp.dot(x8b, centT.astype(jnp.bfloat16),
                    preferred_element_type=jnp.float32)
        c2 = jnp.sum(centT * centT, axis=0, keepdims=True)
        d2 = (x2 - 2.0 * t) + c2
        m = jnp.min(d2, axis=1, keepdims=True)
        labels = jnp.min(jnp.where(d2 == m, iota_k, K), axis=1)
        labels = labels.astype(jnp.int32)
        oh = (labels[:, None] == iota_k).astype(jnp.bfloat16)
        statsT = jnp.dot(x8tb, oh,
                         preferred_element_type=jnp.float32)
        counts = jnp.sum(statsT * e3_c, axis=0, keepdims=True)
        centT_new = statsT * cmask_c / jnp.maximum(counts, 1.0)
        return (centT_new, labels)

    labels0 = jnp.zeros((N,), dtype=jnp.int32)
    centT, labels = lax.fori_loop(0, ITERS, one_iter, (centT0, labels0))

    oh = (labels[:, None] == iota_k).astype(jnp.float32)
    cent8 = centT.T
    p_i8 = jnp.dot(oh, cent8, preferred_element_type=jnp.float32,
                   precision=lax.Precision.HIGHEST)
    rel8 = x8 * cmask_r - p_i8

    h1 = jnp.dot(rel8, w1_ref[...], preferred_element_type=jnp.float32)
    h1 = h1 + b1_ref[...]
    mu = jnp.mean(h1, axis=1, keepdims=True)
    var = jnp.mean((h1 - mu) ** 2, axis=1, keepdims=True)
    hn = (h1 - mu) / jnp.sqrt(var + 1e-5) * g1_ref[...] + be1_ref[...]
    h = jnp.dot(jnp.maximum(hn, 0.0), w2_ref[...],
                preferred_element_type=jnp.float32) + b2_ref[...]

    cent_ref[0] = cent8[:, :3]
    lab_ref[0, 0] = labels
    pi_ref[0] = p_i8[:, :3]
    h_ref[0] = h


def _stage1(x8, x8t, w1p, b1, g1, be1, w2, b2):
    return pl.pallas_call(
        _stage1_body,
        grid=(B,),
        in_specs=[
            pl.BlockSpec((1, N, 8), lambda b: (b, 0, 0)),
            pl.BlockSpec((1, 8, N), lambda b: (b, 0, 0)),
            pl.BlockSpec((8, D1), lambda b: (0, 0)),
            pl.BlockSpec((1, D1), lambda b: (0, 0)),
            pl.BlockSpec((1, D1), lambda b: (0, 0)),
            pl.BlockSpec((1, D1), lambda b: (0, 0)),
            pl.BlockSpec((D1, D2), lambda b: (0, 0)),
            pl.BlockSpec((1, D2), lambda b: (0, 0)),
        ],
        out_specs=[
            pl.BlockSpec((1, K, 3), lambda b: (b, 0, 0)),
            pl.BlockSpec((1, 1, N), lambda b: (b, 0, 0)),
            pl.BlockSpec((1, N, 3), lambda b: (b, 0, 0)),
            pl.BlockSpec((1, N, D2), lambda b: (b, 0, 0)),
        ],
        out_shape=[
            jax.ShapeDtypeStruct((B, K, 3), jnp.float32),
            jax.ShapeDtypeStruct((B, 1, N), jnp.int32),
            jax.ShapeDtypeStruct((B, N, 3), jnp.float32),
            jax.ShapeDtypeStruct((B, N, D2), jnp.float32),
        ],
        compiler_params=pltpu.CompilerParams(
            dimension_semantics=("arbitrary",)),
    )(x8, x8t, w1p, b1, g1, be1, w2, b2)


def _segmax_sc(data, labels, packed=False):
    mesh = plsc.VectorSubcoreMesh(core_axis_name="c", subcore_axis_name="s")
    if packed:
        half = data.shape[-1]
        out_ty = jax.ShapeDtypeStruct((B, 2, K, half), jnp.float32)
    else:
        half = data.shape[-1] // 2
        out_ty = jax.ShapeDtypeStruct((B, K, data.shape[-1]), jnp.float32)

    if packed:
        def vmax(a, v):
            return plsc.bitcast(
                jnp.maximum(plsc.bitcast(a, jnp.bfloat16),
                            plsc.bitcast(v, jnp.bfloat16)), jnp.float32)

        def vfix(v):
            vu = plsc.bitcast(v, jnp.uint32)
            negw = jnp.full((16,), 0xFF80FF80, dtype=jnp.uint32)
            return jnp.where(vu == negw, jnp.zeros((16,), jnp.float32), v)

        def vneg():
            return plsc.bitcast(
                jnp.full((16,), 0xFF80FF80, dtype=jnp.uint32), jnp.float32)
    else:
        def vmax(a, v):
            return jnp.maximum(a, v)

        def vfix(v):
            return jnp.where(v == -jnp.inf, 0.0, v)

        def vneg():
            return jnp.full((16,), -jnp.inf, dtype=jnp.float32)

    @functools.partial(
        pl.kernel,
        out_type=out_ty,
        mesh=mesh,
        scratch_types=[
            pltpu.VMEM((N,), jnp.int32),
            pltpu.VMEM((CH, half), jnp.float32),
            pltpu.VMEM((K, half), jnp.float32),
            pltpu.VMEM((K, half), jnp.float32),
        ],
    )
    def seg(data_hbm, lab_hbm, out_hbm, lab_v, buf, acc0, acc1):
        b = lax.axis_index("s")
        ch = lax.axis_index("c")
        col0 = ch * half
        pltpu.sync_copy(lab_hbm.at[b], lab_v)

        neg = vneg()

        def init_body(k, carry):
            for j in range(half // 16):
                acc0[k, pl.ds(j * 16, 16)] = neg
                acc1[k, pl.ds(j * 16, 16)] = neg
            return carry
        lax.fori_loop(0, K, init_body, 0)

        def chunk_body(g, carry):
            r0 = g * CH
            if packed:
                pltpu.sync_copy(data_hbm.at[b, ch, pl.ds(r0, CH), :], buf)
            else:
                pltpu.sync_copy(
                    data_hbm.at[b, pl.ds(r0, CH), pl.ds(col0, half)], buf)

            def row_body(rg, c2):
                lv = lab_v[pl.ds(r0 + rg * 16, 16)]
                for i in range(0, 16, 2):
                    l0 = lv[i]
                    l1 = lv[i + 1]
                    r = rg * 16 + i
                    for j in range(half // 16):
                        a = acc0[l0, pl.ds(j * 16, 16)]
                        v = buf[r, pl.ds(j * 16, 16)]
                        acc0[l0, pl.ds(j * 16, 16)] = vmax(a, v)
                        a1 = acc1[l1, pl.ds(j * 16, 16)]
                        v1 = buf[r + 1, pl.ds(j * 16, 16)]
                        acc1[l1, pl.ds(j * 16, 16)] = vmax(a1, v1)
                return c2
            lax.fori_loop(0, CH // 16, row_body, 0)
            return carry
        lax.fori_loop(0, N // CH, chunk_body, 0)

        def fix_body(k, carry):
            for j in range(half // 16):
                v = vmax(acc0[k, pl.ds(j * 16, 16)],
                         acc1[k, pl.ds(j * 16, 16)])
                acc0[k, pl.ds(j * 16, 16)] = vfix(v)
            return carry
        lax.fori_loop(0, K, fix_body, 0)

        if packed:
            pltpu.sync_copy(acc0, out_hbm.at[b, ch])
        else:
            pltpu.sync_copy(acc0, out_hbm.at[b, :, pl.ds(col0, half)])

    return seg(data, labels)


def _stage2_body(h_ref, lab_ref, pool_ref, w3a_ref, w3b_ref,
                 b3_ref, g2_ref, be2_ref, w4_ref, b4_ref, h2_ref):
    labels = lab_ref[0, 0]
    iota_k = lax.broadcasted_iota(jnp.int32, (1, K), 1)
    oh = (labels[:, None] == iota_k).astype(jnp.bfloat16)
    reapted = jnp.dot(oh, pool_ref[0].astype(jnp.bfloat16),
                      preferred_element_type=jnp.float32)
    z = (jnp.dot(reapted, w3a_ref[...], preferred_element_type=jnp.float32)
         + jnp.dot(h_ref[0], w3b_ref[...], preferred_element_type=jnp.float32)
         + b3_ref[...])
    mu = jnp.mean(z, axis=1, keepdims=True)
    var = jnp.mean((z - mu) ** 2, axis=1, keepdims=True)
    zn = (z - mu) / jnp.sqrt(var + 1e-5) * g2_ref[...] + be2_ref[...]
    h2 = jnp.dot(jnp.maximum(zn, 0.0), w4_ref[...],
                 preferred_element_type=jnp.float32) + b4_ref[...]
    h2_ref[0] = h2


def _stage2(h, labels3, pooled, w3a, w3b, b3, g2, be2, w4, b4):
    return pl.pallas_call(
        _stage2_body,
        grid=(B,),
        in_specs=[
            pl.BlockSpec((1, N, D2), lambda b: (b, 0, 0)),
            pl.BlockSpec((1, 1, N), lambda b: (b, 0, 0)),
            pl.BlockSpec((1, K, D2), lambda b: (b, 0, 0)),
            pl.BlockSpec((D2, D3), lambda b: (0, 0)),
            pl.BlockSpec((D2, D3), lambda b: (0, 0)),
            pl.BlockSpec((1, D3), lambda b: (0, 0)),
            pl.BlockSpec((1, D3), lambda b: (0, 0)),
            pl.BlockSpec((1, D3), lambda b: (0, 0)),
            pl.BlockSpec((D3, D4), lambda b: (0, 0)),
            pl.BlockSpec((1, D4), lambda b: (0, 0)),
        ],
        out_specs=pl.BlockSpec((1, N, D4), lambda b: (b, 0, 0)),
        out_shape=jax.ShapeDtypeStruct((B, N, D4), jnp.float32),
        compiler_params=pltpu.CompilerParams(
            dimension_semantics=("arbitrary",)),
    )(h, labels3, pooled, w3a, w3b, b3, g2, be2, w4, b4)


def kernel(xyz, W1, b1, g1, beta1, W2, b2, W3, b3, g2, beta2, W4, b4):
    ones = jnp.ones((B, N, 1), jnp.float32)
    zeros = jnp.zeros((B, N, 4), jnp.float32)
    x8 = jnp.concatenate([xyz, ones, zeros], axis=-1)
    x8t = jnp.transpose(x8, (0, 2, 1))
    w1p = jnp.concatenate([W1, jnp.zeros((5, D1), jnp.float32)], axis=0)

    cent, labels3, p_i, h = _stage1(
        x8, x8t, w1p, b1.reshape(1, D1), g1.reshape(1, D1),
        beta1.reshape(1, D1), W2, b2.reshape(1, D2))

    labels = labels3.reshape(B, N)
    pooled = _segmax_sc(h, labels)

    h2 = _stage2(h, labels3, pooled, W3[:D2], W3[D2:], b3.reshape(1, D3),
                 g2.reshape(1, D3), beta2.reshape(1, D3), W4,
                 b4.reshape(1, D4))

    out = _segmax_sc(h2, labels)
    return (cent, out, p_i, labels)


# speedup vs baseline: 247.0963x; 1.1034x over previous
import functools

import jax
import jax.numpy as jnp
from jax import lax
from jax.experimental import pallas as pl
from jax.experimental.pallas import tpu as pltpu
from jax.experimental.pallas import tpu_sc as plsc

B, N, K, ITERS = 16, 4096, 256, 10
D1, D2, D3, D4 = 128, 256, 512, 256
HALF = D2 // 2
CH = 256


def _stage1_body(x8_ref, x8t_ref, w1_ref, b1_ref, g1_ref, be1_ref,
                 w2_ref, b2_ref, cent_ref, lab_ref, pi_ref, h_ref):
    x8 = x8_ref[0]
    x8t = x8t_ref[0]
    iota8r = lax.broadcasted_iota(jnp.int32, (1, 8), 1)
    iota8c = lax.broadcasted_iota(jnp.int32, (8, 1), 0)
    cmask_r = (iota8r < 3).astype(jnp.float32)
    cmask_c = (iota8c < 3).astype(jnp.float32)
    e3_c = (iota8c == 3).astype(jnp.float32)
    x2 = jnp.sum(x8 * x8 * cmask_r, axis=1, keepdims=True)
    iota_k = lax.broadcasted_iota(jnp.int32, (1, K), 1)

    centT0 = (x8[:K, :] * cmask_r).T
    x8b = x8.astype(jnp.bfloat16)
    x8tb = x8t.astype(jnp.bfloat16)

    def one_iter(_, carry):
        centT, _labels = carry
        t = jnp.dot(x8b, centT.astype(jnp.bfloat16),
                    preferred_element_type=jnp.float32)
        c2 = jnp.sum(centT * centT, axis=0, keepdims=True)
        d2 = (x2 - 2.0 * t) + c2
        m = jnp.min(d2, axis=1, keepdims=True)
        labels = jnp.min(jnp.where(d2 == m, iota_k, K), axis=1)
        labels = labels.astype(jnp.int32)
        oh = (labels[:, None] == iota_k).astype(jnp.bfloat16)
        statsT = jnp.dot(x8tb, oh,
                         preferred_element_type=jnp.float32)
        counts = jnp.sum(statsT * e3_c, axis=0, keepdims=True)
        centT_new = statsT * cmask_c / jnp.maximum(counts, 1.0)
        return (centT_new, labels)

    labels0 = jnp.zeros((N,), dtype=jnp.int32)
    centT, labels = lax.fori_loop(0, ITERS, one_iter, (centT0, labels0))

    oh = (labels[:, None] == iota_k).astype(jnp.float32)
    cent8 = centT.T
    p_i8 = jnp.dot(oh, cent8, preferred_element_type=jnp.float32,
                   precision=lax.Precision.HIGHEST)
    rel8 = x8 * cmask_r - p_i8

    h1 = jnp.dot(rel8, w1_ref[...], preferred_element_type=jnp.float32)
    h1 = h1 + b1_ref[...]
    mu = jnp.mean(h1, axis=1, keepdims=True)
    var = jnp.mean((h1 - mu) ** 2, axis=1, keepdims=True)
    hn = (h1 - mu) / jnp.sqrt(var + 1e-5) * g1_ref[...] + be1_ref[...]
    h = jnp.dot(jnp.maximum(hn, 0.0), w2_ref[...],
                preferred_element_type=jnp.float32) + b2_ref[...]

    cent_ref[0] = cent8[:, :3]
    lab_ref[0, 0] = labels
    pi_ref[0] = p_i8[:, :3]
    h_ref[0] = h


def _stage1(x8, x8t, w1p, b1, g1, be1, w2, b2):
    return pl.pallas_call(
        _stage1_body,
        grid=(B,),
        in_specs=[
            pl.BlockSpec((1, N, 8), lambda b: (b, 0, 0)),
            pl.BlockSpec((1, 8, N), lambda b: (b, 0, 0)),
            pl.BlockSpec((8, D1), lambda b: (0, 0)),
            pl.BlockSpec((1, D1), lambda b: (0, 0)),
            pl.BlockSpec((1, D1), lambda b: (0, 0)),
            pl.BlockSpec((1, D1), lambda b: (0, 0)),
            pl.BlockSpec((D1, D2), lambda b: (0, 0)),
            pl.BlockSpec((1, D2), lambda b: (0, 0)),
        ],
        out_specs=[
            pl.BlockSpec((1, K, 3), lambda b: (b, 0, 0)),
            pl.BlockSpec((1, 1, N), lambda b: (b, 0, 0)),
            pl.BlockSpec((1, N, 3), lambda b: (b, 0, 0)),
            pl.BlockSpec((1, N, D2), lambda b: (b, 0, 0)),
        ],
        out_shape=[
            jax.ShapeDtypeStruct((B, K, 3), jnp.float32),
            jax.ShapeDtypeStruct((B, 1, N), jnp.int32),
            jax.ShapeDtypeStruct((B, N, 3), jnp.float32),
            jax.ShapeDtypeStruct((B, N, D2), jnp.float32),
        ],
        compiler_params=pltpu.CompilerParams(
            dimension_semantics=("arbitrary",)),
    )(x8, x8t, w1p, b1, g1, be1, w2, b2)


def _segmax_sc(data, labels, packed=False):
    mesh = plsc.VectorSubcoreMesh(core_axis_name="c", subcore_axis_name="s")
    if packed:
        half = data.shape[-1]
        out_ty = jax.ShapeDtypeStruct((B, 2, K, half), jnp.float32)
    else:
        half = data.shape[-1] // 2
        out_ty = jax.ShapeDtypeStruct((B, K, data.shape[-1]), jnp.float32)

    if packed:
        def vmax(a, v):
            return plsc.bitcast(
                jnp.maximum(plsc.bitcast(a, jnp.bfloat16),
                            plsc.bitcast(v, jnp.bfloat16)), jnp.float32)

        def vfix(v):
            vu = plsc.bitcast(v, jnp.uint32)
            negw = jnp.full((16,), 0xFF80FF80, dtype=jnp.uint32)
            return jnp.where(vu == negw, jnp.zeros((16,), jnp.float32), v)

        def vneg():
            return plsc.bitcast(
                jnp.full((16,), 0xFF80FF80, dtype=jnp.uint32), jnp.float32)
    else:
        def vmax(a, v):
            return jnp.maximum(a, v)

        def vfix(v):
            return jnp.where(v == -jnp.inf, 0.0, v)

        def vneg():
            return jnp.full((16,), -jnp.inf, dtype=jnp.float32)

    @functools.partial(
        pl.kernel,
        out_type=out_ty,
        mesh=mesh,
        scratch_types=[
            pltpu.VMEM((N,), jnp.int32),
            pltpu.VMEM((CH, half), jnp.float32),
            pltpu.VMEM((CH, half), jnp.float32),
            pltpu.VMEM((K, half), jnp.float32),
            pltpu.SemaphoreType.DMA,
            pltpu.SemaphoreType.DMA,
        ],
    )
    def seg(data_hbm, lab_hbm, out_hbm, lab_v, buf0, buf1, acc0, sem0, sem1):
        b = lax.axis_index("s")
        ch = lax.axis_index("c")
        col0 = ch * half

        def src(g):
            if packed:
                return data_hbm.at[b, ch, pl.ds(g * CH, CH), :]
            return data_hbm.at[b, pl.ds(g * CH, CH), pl.ds(col0, half)]

        pltpu.sync_copy(lab_hbm.at[b], lab_v)

        neg = vneg()

        def init_body(k, carry):
            for j in range(half // 16):
                acc0[k, pl.ds(j * 16, 16)] = neg
            return carry
        lax.fori_loop(0, K, init_body, 0)

        def process(buf, r0):
            def row_body(rg, c2):
                lv = lab_v[pl.ds(r0 + rg * 16, 16)]
                for i in range(16):
                    l0 = lv[i]
                    r = rg * 16 + i
                    for j in range(half // 16):
                        a = acc0[l0, pl.ds(j * 16, 16)]
                        v = buf[r, pl.ds(j * 16, 16)]
                        acc0[l0, pl.ds(j * 16, 16)] = vmax(a, v)
                return c2
            lax.fori_loop(0, CH // 16, row_body, 0)

        pltpu.async_copy(src(0), buf0, sem0)
        nch = N // CH

        def outer(gp, carry):
            g0 = 2 * gp
            pltpu.make_async_copy(src(g0), buf0, sem0).wait()
            pltpu.async_copy(src(g0 + 1), buf1, sem1)
            process(buf0, g0 * CH)
            pltpu.make_async_copy(src(g0 + 1), buf1, sem1).wait()

            @pl.when(gp < nch // 2 - 1)
            def _():
                pltpu.async_copy(src(g0 + 2), buf0, sem0)
            process(buf1, (g0 + 1) * CH)
            return carry
        lax.fori_loop(0, nch // 2, outer, 0)

        def fix_body(k, carry):
            for j in range(half // 16):
                v = acc0[k, pl.ds(j * 16, 16)]
                acc0[k, pl.ds(j * 16, 16)] = vfix(v)
            return carry
        lax.fori_loop(0, K, fix_body, 0)

        if packed:
            pltpu.sync_copy(acc0, out_hbm.at[b, ch])
        else:
            pltpu.sync_copy(acc0, out_hbm.at[b, :, pl.ds(col0, half)])

    return seg(data, labels)


def _stage2_body(h_ref, lab_ref, pool_ref, w3a_ref, w3b_ref,
                 b3_ref, g2_ref, be2_ref, w4_ref, b4_ref, h2_ref):
    labels = lab_ref[0, 0]
    iota_k = lax.broadcasted_iota(jnp.int32, (1, K), 1)
    oh = (labels[:, None] == iota_k).astype(jnp.bfloat16)
    reapted = jnp.dot(oh, pool_ref[0].astype(jnp.bfloat16),
                      preferred_element_type=jnp.float32)
    z = (jnp.dot(reapted, w3a_ref[...], preferred_element_type=jnp.float32)
         + jnp.dot(h_ref[0], w3b_ref[...], preferred_element_type=jnp.float32)
         + b3_ref[...])
    mu = jnp.mean(z, axis=1, keepdims=True)
    var = jnp.mean((z - mu) ** 2, axis=1, keepdims=True)
    zn = (z - mu) / jnp.sqrt(var + 1e-5) * g2_ref[...] + be2_ref[...]
    h2 = jnp.dot(jnp.maximum(zn, 0.0), w4_ref[...],
                 preferred_element_type=jnp.float32) + b4_ref[...]
    h2_ref[0] = h2


def _stage2(h, labels3, pooled, w3a, w3b, b3, g2, be2, w4, b4):
    return pl.pallas_call(
        _stage2_body,
        grid=(B,),
        in_specs=[
            pl.BlockSpec((1, N, D2), lambda b: (b, 0, 0)),
            pl.BlockSpec((1, 1, N), lambda b: (b, 0, 0)),
            pl.BlockSpec((1, K, D2), lambda b: (b, 0, 0)),
            pl.BlockSpec((D2, D3), lambda b: (0, 0)),
            pl.BlockSpec((D2, D3), lambda b: (0, 0)),
            pl.BlockSpec((1, D3), lambda b: (0, 0)),
            pl.BlockSpec((1, D3), lambda b: (0, 0)),
            pl.BlockSpec((1, D3), lambda b: (0, 0)),
            pl.BlockSpec((D3, D4), lambda b: (0, 0)),
            pl.BlockSpec((1, D4), lambda b: (0, 0)),
        ],
        out_specs=pl.BlockSpec((1, N, D4), lambda b: (b, 0, 0)),
        out_shape=jax.ShapeDtypeStruct((B, N, D4), jnp.float32),
        compiler_params=pltpu.CompilerParams(
            dimension_semantics=("arbitrary",)),
    )(h, labels3, pooled, w3a, w3b, b3, g2, be2, w4, b4)


def kernel(xyz, W1, b1, g1, beta1, W2, b2, W3, b3, g2, beta2, W4, b4):
    ones = jnp.ones((B, N, 1), jnp.float32)
    zeros = jnp.zeros((B, N, 4), jnp.float32)
    x8 = jnp.concatenate([xyz, ones, zeros], axis=-1)
    x8t = jnp.transpose(x8, (0, 2, 1))
    w1p = jnp.concatenate([W1, jnp.zeros((5, D1), jnp.float32)], axis=0)

    cent, labels3, p_i, h = _stage1(
        x8, x8t, w1p, b1.reshape(1, D1), g1.reshape(1, D1),
        beta1.reshape(1, D1), W2, b2.reshape(1, D2))

    labels = labels3.reshape(B, N)
    pooled = _segmax_sc(h, labels)

    h2 = _stage2(h, labels3, pooled, W3[:D2], W3[D2:], b3.reshape(1, D3),
                 g2.reshape(1, D3), beta2.reshape(1, D3), W4,
                 b4.reshape(1, D4))

    out = _segmax_sc(h2, labels)
    return (cent, out, p_i, labels)
